# baseline (speedup 1.0000x reference)
import jax
import jax.numpy as jnp
from jax import lax
from jax.experimental import pallas as pl
from jax.experimental.pallas import tpu as pltpu


def _linear_t_body(xt_ref, w_ref, b_ref, o_ref):
    acc = lax.dot_general(
        w_ref[...], xt_ref[...],
        dimension_numbers=(((1,), (0,)), ((), ())),
        preferred_element_type=jnp.float32,
    )
    o_ref[...] = (acc + b_ref[...]).astype(o_ref.dtype)


def kernel(x, weight, bias, block_cols=32768):
    B, IN = x.shape
    OUT, IN_w = weight.shape
    assert IN == IN_w
    bn = max(128, min(block_cols, B))
    xt = x.T
    yt = pl.pallas_call(
        _linear_t_body,
        out_shape=jax.ShapeDtypeStruct((OUT, B), x.dtype),
        grid=(pl.cdiv(B, bn),),
        in_specs=[
            pl.BlockSpec((IN, bn), lambda i: (0, i)),
            pl.BlockSpec((OUT, IN), lambda i: (0, 0)),
            pl.BlockSpec((OUT, 1), lambda i: (0, 0)),
        ],
        out_specs=pl.BlockSpec((OUT, bn), lambda i: (0, i)),
        compiler_params=pltpu.CompilerParams(
            dimension_semantics=("parallel",),
        ),
    )(xt, weight, bias.reshape(OUT, 1))
    return yt.T


# speedup vs baseline: 1.0416x; 1.0416x over previous
import jax
import jax.numpy as jnp
from jax import lax
from jax.experimental import pallas as pl
from jax.experimental.pallas import tpu as pltpu


def _linear_t_body(xt_ref, w_ref, b_ref, o_ref):
    acc = lax.dot_general(
        w_ref[...], xt_ref[...],
        dimension_numbers=(((1,), (0,)), ((), ())),
        preferred_element_type=jnp.float32,
    )
    o_ref[...] = (acc + b_ref[...]).astype(o_ref.dtype)


def kernel(x, weight, bias, block_cols=65536):
    B, IN = x.shape
    OUT, IN_w = weight.shape
    assert IN == IN_w
    bn = max(128, min(block_cols, B))
    xt = x.T
    yt = pl.pallas_call(
        _linear_t_body,
        out_shape=jax.ShapeDtypeStruct((OUT, B), x.dtype),
        grid=(pl.cdiv(B, bn),),
        in_specs=[
            pl.BlockSpec((IN, bn), lambda i: (0, i)),
            pl.BlockSpec((OUT, IN), lambda i: (0, 0)),
            pl.BlockSpec((OUT, 1), lambda i: (0, 0)),
        ],
        out_specs=pl.BlockSpec((OUT, bn), lambda i: (0, i)),
        compiler_params=pltpu.CompilerParams(
            dimension_semantics=("parallel",),
        ),
    )(xt, weight, bias.reshape(OUT, 1))
    return yt.T
